# Initial kernel scaffold; baseline (speedup 1.0000x reference)
import jax
import jax.numpy as jnp
from jax.experimental import pallas as pl


def kernel(pred_logits, pred_boxes, target_sizes, labels):
    raise NotImplementedError("write your pallas kernel here")


# speedup vs baseline: 804.5021x; 804.5021x over previous
import jax
import jax.numpy as jnp
from jax.experimental import pallas as pl
from jax.experimental.pallas import tpu as pltpu

_B = 4
_NUM_PATCH = 10
_NUM_QUERIES = 500
_P = _B * _NUM_PATCH
_PRE = 100
_KEEP = 20
_IOU_TH = 0.7


def _bid(x, shape, dims):
    return jax.lax.broadcast_in_dim(x, shape, dims)


def _nms_body(sl_ref, cx_ref, cy_ref, w_ref, h_ref, sw_ref, sh_ref, lab_ref,
              os_ref, ox1_ref, oy1_ref, ox2_ref, oy2_ref, ol_ref, sup_ref):
    f32 = jnp.float32
    P, NQ, PRE, KEEP = _P, _NUM_QUERIES, _PRE, _KEEP
    s = jax.nn.sigmoid(sl_ref[...])
    sw = sw_ref[...]
    sh = sh_ref[...]
    cx = cx_ref[...]
    cy = cy_ref[...]
    w = w_ref[...]
    h = h_ref[...]
    x1 = (cx - 0.5 * w) * sw
    y1 = (cy - 0.5 * h) * sh
    x2 = (cx + 0.5 * w) * sw
    y2 = (cy + 0.5 * h) * sh

    st = jnp.transpose(s)
    rank = jnp.zeros((P, NQ), f32)
    CH = 50
    for c0 in range(0, NQ, CH):
        sj3 = _bid(st[c0:c0 + CH, :], (CH, P, NQ), (0, 1))
        si3 = _bid(s, (CH, P, NQ), (1, 2))
        jj3 = jax.lax.broadcasted_iota(jnp.int32, (CH, P, NQ), 0) + c0
        ii3 = jax.lax.broadcasted_iota(jnp.int32, (CH, P, NQ), 2)
        cmp = (sj3 > si3) | ((sj3 == si3) & (jj3 < ii3))
        rank = rank + jnp.sum(jnp.where(cmp, 1.0, 0.0), axis=0)

    sp_c, x1_c, y1_c, x2_c, y2_c = [], [], [], [], []
    RCH = 50
    for r0 in range(0, PRE, RCH):
        rr = (jax.lax.broadcasted_iota(jnp.int32, (RCH, P, NQ), 0)
              + r0).astype(f32)
        oh = _bid(rank, (RCH, P, NQ), (1, 2)) == rr
        z = jnp.zeros((RCH, P, NQ), f32)
        sp_c.append(jnp.transpose(jnp.sum(
            jnp.where(oh, _bid(s, (RCH, P, NQ), (1, 2)), z), axis=2)))
        x1_c.append(jnp.transpose(jnp.sum(
            jnp.where(oh, _bid(x1, (RCH, P, NQ), (1, 2)), z), axis=2)))
        y1_c.append(jnp.transpose(jnp.sum(
            jnp.where(oh, _bid(y1, (RCH, P, NQ), (1, 2)), z), axis=2)))
        x2_c.append(jnp.transpose(jnp.sum(
            jnp.where(oh, _bid(x2, (RCH, P, NQ), (1, 2)), z), axis=2)))
        y2_c.append(jnp.transpose(jnp.sum(
            jnp.where(oh, _bid(y2, (RCH, P, NQ), (1, 2)), z), axis=2)))
    sp = jnp.concatenate(sp_c, axis=1)
    bx1 = jnp.concatenate(x1_c, axis=1)
    by1 = jnp.concatenate(y1_c, axis=1)
    bx2 = jnp.concatenate(x2_c, axis=1)
    by2 = jnp.concatenate(y2_c, axis=1)

    labf = lab_ref[...].astype(f32)
    m = jnp.maximum(jnp.maximum(jnp.max(bx1, axis=1, keepdims=True),
                                jnp.max(by1, axis=1, keepdims=True)),
                    jnp.maximum(jnp.max(bx2, axis=1, keepdims=True),
                                jnp.max(by2, axis=1, keepdims=True)))
    off = labf * (m + 1.0)
    qx1 = bx1 + off
    qy1 = by1 + off
    qx2 = bx2 + off
    qy2 = by2 + off

    shp = (PRE, P, PRE)
    X1 = jnp.maximum(_bid(jnp.transpose(qx1), shp, (0, 1)),
                     _bid(qx1, shp, (1, 2)))
    Y1 = jnp.maximum(_bid(jnp.transpose(qy1), shp, (0, 1)),
                     _bid(qy1, shp, (1, 2)))
    X2 = jnp.minimum(_bid(jnp.transpose(qx2), shp, (0, 1)),
                     _bid(qx2, shp, (1, 2)))
    Y2 = jnp.minimum(_bid(jnp.transpose(qy2), shp, (0, 1)),
                     _bid(qy2, shp, (1, 2)))
    inter = jnp.maximum(X2 - X1, 0.0) * jnp.maximum(Y2 - Y1, 0.0)
    a = (qx2 - qx1) * (qy2 - qy1)
    iou = inter / (_bid(jnp.transpose(a), shp, (0, 1))
                   + _bid(a, shp, (1, 2)) - inter + 1e-9)
    sup_ref[...] = jnp.where(iou > _IOU_TH, 1.0, 0.0)

    lane = jax.lax.broadcasted_iota(jnp.int32, (P, PRE), 1)

    def body(i, keep):
        row = jnp.reshape(sup_ref[pl.ds(i, 1), :, :], (P, PRE))
        onei = jnp.where(lane == i, 1.0, 0.0)
        kp_i = jnp.sum(keep * onei, axis=1, keepdims=True)
        gtmask = jnp.where(lane > i, 1.0, 0.0)
        return keep * (1.0 - row * gtmask * kp_i)

    keep = jax.lax.fori_loop(0, PRE, body, jnp.ones((P, PRE), f32))

    tri = jnp.where(jax.lax.broadcasted_iota(jnp.int32, (PRE, PRE), 0)
                    <= jax.lax.broadcasted_iota(jnp.int32, (PRE, PRE), 1),
                    1.0, 0.0)
    kc = jax.lax.dot_general(keep, tri, (((1,), (0,)), ((), ())),
                             preferred_element_type=f32)
    nkept = jnp.sum(keep, axis=1, keepdims=True)
    kshp = (KEEP, P, PRE)
    rl3 = jax.lax.broadcasted_iota(jnp.int32, kshp, 0).astype(f32)
    oh20 = ((_bid(kc, kshp, (1, 2)) - 1.0) == rl3) & (_bid(keep, kshp, (1, 2)) > 0.0)
    zk = jnp.zeros(kshp, f32)
    os_ref[...] = jnp.transpose(jnp.sum(
        jnp.where(oh20, _bid(sp, kshp, (1, 2)), zk), axis=2))
    ox1_ref[...] = jnp.transpose(jnp.sum(
        jnp.where(oh20, _bid(bx1, kshp, (1, 2)), zk), axis=2))
    oy1_ref[...] = jnp.transpose(jnp.sum(
        jnp.where(oh20, _bid(by1, kshp, (1, 2)), zk), axis=2))
    ox2_ref[...] = jnp.transpose(jnp.sum(
        jnp.where(oh20, _bid(bx2, kshp, (1, 2)), zk), axis=2))
    oy2_ref[...] = jnp.transpose(jnp.sum(
        jnp.where(oh20, _bid(by2, kshp, (1, 2)), zk), axis=2))
    rlane = jax.lax.broadcasted_iota(jnp.int32, (P, KEEP), 1).astype(f32)
    valid = rlane < nkept
    ol_ref[...] = jnp.where(valid, lab_ref[...], -1)


def kernel(pred_logits, pred_boxes, target_sizes, labels):
    f32 = jnp.float32
    sl = pred_logits[:, :, 1].reshape(_P, _NUM_QUERIES)
    cx = pred_boxes[:, :, 0].reshape(_P, _NUM_QUERIES)
    cy = pred_boxes[:, :, 1].reshape(_P, _NUM_QUERIES)
    w = pred_boxes[:, :, 2].reshape(_P, _NUM_QUERIES)
    h = pred_boxes[:, :, 3].reshape(_P, _NUM_QUERIES)
    ts = target_sizes.astype(f32)
    sh = jnp.repeat(ts[:, 0], _NUM_PATCH).reshape(_P, 1)
    sw = jnp.repeat(ts[:, 1], _NUM_PATCH).reshape(_P, 1)
    lab = labels.astype(jnp.int32).reshape(_P, 1)

    out_shape = [
        jax.ShapeDtypeStruct((_P, _KEEP), f32),
        jax.ShapeDtypeStruct((_P, _KEEP), f32),
        jax.ShapeDtypeStruct((_P, _KEEP), f32),
        jax.ShapeDtypeStruct((_P, _KEEP), f32),
        jax.ShapeDtypeStruct((_P, _KEEP), f32),
        jax.ShapeDtypeStruct((_P, _KEEP), jnp.int32),
    ]
    os_, x1o, y1o, x2o, y2o, ol = pl.pallas_call(
        _nms_body,
        out_shape=out_shape,
        scratch_shapes=[pltpu.VMEM((_PRE, _P, _PRE), f32)],
    )(sl, cx, cy, w, h, sw, sh, lab)

    s_out = os_.reshape(_P * _KEEP)
    b_out = jnp.stack([x1o, y1o, x2o, y2o], axis=-1).reshape(_P * _KEEP, 4)
    l_out = ol.reshape(_P * _KEEP)
    return (s_out, b_out, l_out)


# speedup vs baseline: 860.4965x; 1.0696x over previous
import jax
import jax.numpy as jnp
from jax.experimental import pallas as pl
from jax.experimental.pallas import tpu as pltpu

_B = 4
_NUM_PATCH = 10
_NUM_QUERIES = 500
_P = _B * _NUM_PATCH
_PRE = 100
_KEEP = 20
_IOU_TH = 0.7


def _bid(x, shape, dims):
    return jax.lax.broadcast_in_dim(x, shape, dims)


def _nms_body(lg_ref, bx_ref, sw_ref, sh_ref, lab_ref,
              os_ref, ox1_ref, oy1_ref, ox2_ref, oy2_ref, ol_ref, sup_ref):
    f32 = jnp.float32
    P, NQ, PRE, KEEP = _P, _NUM_QUERIES, _PRE, _KEEP
    lg = lg_ref[...]
    bx = bx_ref[...]
    s = jax.nn.sigmoid(jnp.reshape(lg[:, 1:2, :], (P, NQ)))
    sw = sw_ref[...]
    sh = sh_ref[...]
    cx = jnp.reshape(bx[:, 0:1, :], (P, NQ))
    cy = jnp.reshape(bx[:, 1:2, :], (P, NQ))
    w = jnp.reshape(bx[:, 2:3, :], (P, NQ))
    h = jnp.reshape(bx[:, 3:4, :], (P, NQ))
    x1 = (cx - 0.5 * w) * sw
    y1 = (cy - 0.5 * h) * sh
    x2 = (cx + 0.5 * w) * sw
    y2 = (cy + 0.5 * h) * sh

    st = jnp.transpose(s)
    rank = jnp.zeros((P, NQ), f32)
    CH = 50
    for c0 in range(0, NQ, CH):
        sj3 = _bid(st[c0:c0 + CH, :], (CH, P, NQ), (0, 1))
        si3 = _bid(s, (CH, P, NQ), (1, 2))
        jj3 = jax.lax.broadcasted_iota(jnp.int32, (CH, P, NQ), 0) + c0
        ii3 = jax.lax.broadcasted_iota(jnp.int32, (CH, P, NQ), 2)
        cmp = (sj3 > si3) | ((sj3 == si3) & (jj3 < ii3))
        rank = rank + jnp.sum(jnp.where(cmp, 1.0, 0.0), axis=0)

    sp_c, x1_c, y1_c, x2_c, y2_c = [], [], [], [], []
    RCH = 50
    for r0 in range(0, PRE, RCH):
        rr = (jax.lax.broadcasted_iota(jnp.int32, (RCH, P, NQ), 0)
              + r0).astype(f32)
        oh = _bid(rank, (RCH, P, NQ), (1, 2)) == rr
        z = jnp.zeros((RCH, P, NQ), f32)
        sp_c.append(jnp.transpose(jnp.sum(
            jnp.where(oh, _bid(s, (RCH, P, NQ), (1, 2)), z), axis=2)))
        x1_c.append(jnp.transpose(jnp.sum(
            jnp.where(oh, _bid(x1, (RCH, P, NQ), (1, 2)), z), axis=2)))
        y1_c.append(jnp.transpose(jnp.sum(
            jnp.where(oh, _bid(y1, (RCH, P, NQ), (1, 2)), z), axis=2)))
        x2_c.append(jnp.transpose(jnp.sum(
            jnp.where(oh, _bid(x2, (RCH, P, NQ), (1, 2)), z), axis=2)))
        y2_c.append(jnp.transpose(jnp.sum(
            jnp.where(oh, _bid(y2, (RCH, P, NQ), (1, 2)), z), axis=2)))
    sp = jnp.concatenate(sp_c, axis=1)
    bx1 = jnp.concatenate(x1_c, axis=1)
    by1 = jnp.concatenate(y1_c, axis=1)
    bx2 = jnp.concatenate(x2_c, axis=1)
    by2 = jnp.concatenate(y2_c, axis=1)

    labf = lab_ref[...].astype(f32)
    m = jnp.maximum(jnp.maximum(jnp.max(bx1, axis=1, keepdims=True),
                                jnp.max(by1, axis=1, keepdims=True)),
                    jnp.maximum(jnp.max(bx2, axis=1, keepdims=True),
                                jnp.max(by2, axis=1, keepdims=True)))
    off = labf * (m + 1.0)
    qx1 = bx1 + off
    qy1 = by1 + off
    qx2 = bx2 + off
    qy2 = by2 + off

    shp = (PRE, P, PRE)
    X1 = jnp.maximum(_bid(jnp.transpose(qx1), shp, (0, 1)),
                     _bid(qx1, shp, (1, 2)))
    Y1 = jnp.maximum(_bid(jnp.transpose(qy1), shp, (0, 1)),
                     _bid(qy1, shp, (1, 2)))
    X2 = jnp.minimum(_bid(jnp.transpose(qx2), shp, (0, 1)),
                     _bid(qx2, shp, (1, 2)))
    Y2 = jnp.minimum(_bid(jnp.transpose(qy2), shp, (0, 1)),
                     _bid(qy2, shp, (1, 2)))
    inter = jnp.maximum(X2 - X1, 0.0) * jnp.maximum(Y2 - Y1, 0.0)
    a = (qx2 - qx1) * (qy2 - qy1)
    iou = inter / (_bid(jnp.transpose(a), shp, (0, 1))
                   + _bid(a, shp, (1, 2)) - inter + 1e-9)
    sup_ref[...] = jnp.where(iou > _IOU_TH, 1.0, 0.0)

    lane = jax.lax.broadcasted_iota(jnp.int32, (P, PRE), 1)

    def body(i, keep):
        row = jnp.reshape(sup_ref[pl.ds(i, 1), :, :], (P, PRE))
        onei = jnp.where(lane == i, 1.0, 0.0)
        kp_i = jnp.sum(keep * onei, axis=1, keepdims=True)
        gtmask = jnp.where(lane > i, 1.0, 0.0)
        return keep * (1.0 - row * gtmask * kp_i)

    keep = jax.lax.fori_loop(0, PRE, body, jnp.ones((P, PRE), f32))

    tri = jnp.where(jax.lax.broadcasted_iota(jnp.int32, (PRE, PRE), 0)
                    <= jax.lax.broadcasted_iota(jnp.int32, (PRE, PRE), 1),
                    1.0, 0.0)
    kc = jax.lax.dot_general(keep, tri, (((1,), (0,)), ((), ())),
                             preferred_element_type=f32)
    nkept = jnp.sum(keep, axis=1, keepdims=True)
    kshp = (KEEP, P, PRE)
    rl3 = jax.lax.broadcasted_iota(jnp.int32, kshp, 0).astype(f32)
    oh20 = ((_bid(kc, kshp, (1, 2)) - 1.0) == rl3) & (_bid(keep, kshp, (1, 2)) > 0.0)
    zk = jnp.zeros(kshp, f32)
    os_ref[...] = jnp.transpose(jnp.sum(
        jnp.where(oh20, _bid(sp, kshp, (1, 2)), zk), axis=2))
    ox1_ref[...] = jnp.transpose(jnp.sum(
        jnp.where(oh20, _bid(bx1, kshp, (1, 2)), zk), axis=2))
    oy1_ref[...] = jnp.transpose(jnp.sum(
        jnp.where(oh20, _bid(by1, kshp, (1, 2)), zk), axis=2))
    ox2_ref[...] = jnp.transpose(jnp.sum(
        jnp.where(oh20, _bid(bx2, kshp, (1, 2)), zk), axis=2))
    oy2_ref[...] = jnp.transpose(jnp.sum(
        jnp.where(oh20, _bid(by2, kshp, (1, 2)), zk), axis=2))
    rlane = jax.lax.broadcasted_iota(jnp.int32, (P, KEEP), 1).astype(f32)
    valid = rlane < nkept
    ol_ref[...] = jnp.where(valid, lab_ref[...], -1)


def kernel(pred_logits, pred_boxes, target_sizes, labels):
    f32 = jnp.float32
    lg = jnp.swapaxes(pred_logits.reshape(_P, _NUM_QUERIES, 2), 1, 2)
    bx = jnp.swapaxes(pred_boxes.reshape(_P, _NUM_QUERIES, 4), 1, 2)
    ts = target_sizes.astype(f32)
    sh = jnp.repeat(ts[:, 0], _NUM_PATCH).reshape(_P, 1)
    sw = jnp.repeat(ts[:, 1], _NUM_PATCH).reshape(_P, 1)
    lab = labels.astype(jnp.int32).reshape(_P, 1)

    out_shape = [
        jax.ShapeDtypeStruct((_P, _KEEP), f32),
        jax.ShapeDtypeStruct((_P, _KEEP), f32),
        jax.ShapeDtypeStruct((_P, _KEEP), f32),
        jax.ShapeDtypeStruct((_P, _KEEP), f32),
        jax.ShapeDtypeStruct((_P, _KEEP), f32),
        jax.ShapeDtypeStruct((_P, _KEEP), jnp.int32),
    ]
    os_, x1o, y1o, x2o, y2o, ol = pl.pallas_call(
        _nms_body,
        out_shape=out_shape,
        scratch_shapes=[pltpu.VMEM((_PRE, _P, _PRE), f32)],
    )(lg, bx, sw, sh, lab)

    s_out = os_.reshape(_P * _KEEP)
    b_out = jnp.stack([x1o, y1o, x2o, y2o], axis=-1).reshape(_P * _KEEP, 4)
    l_out = ol.reshape(_P * _KEEP)
    return (s_out, b_out, l_out)
